# baseline (speedup 1.0000x reference)
import functools

import jax
import jax.numpy as jnp
from jax import lax
from jax.experimental import pallas as pl
from jax.experimental.pallas import tpu as pltpu
from jax.experimental.pallas import tpu_sc as plsc

LOG2 = 0.6931471805599453
PI = 3.141592653589793
CUTOFF = 10.0

NC = 2
NS = 16
LANES = 16
NSPLIT = 2


def ssp(v):
    return jnp.maximum(v, 0.0) + jnp.log(1.0 + jnp.exp(-jnp.abs(v))) - LOG2


def _pack_bf16_pairs(y, feat):
    half = feat // 2
    yb = y.astype(jnp.bfloat16)
    lo = lax.bitcast_convert_type(yb[:, :half], jnp.uint16).astype(jnp.uint32)
    hi = lax.bitcast_convert_type(yb[:, half:], jnp.uint16).astype(jnp.uint32)
    return lax.bitcast_convert_type(lo | (hi << 16), jnp.int32)


def _xw_body(x_ref, w_ref, o_ref):
    o_ref[...] = jnp.dot(x_ref[...], w_ref[...],
                         preferred_element_type=jnp.float32)


def _edge_filter_body(eat_ref, ewt_ref, w1_ref, b1_ref, w2_ref, b2_ref, o_ref):
    h1 = lax.dot_general(eat_ref[...], w1_ref[...],
                         (((0,), (0,)), ((), ())),
                         preferred_element_type=jnp.float32) + b1_ref[...]
    h1 = ssp(h1)
    w = jnp.dot(h1, w2_ref[...],
                preferred_element_type=jnp.float32) + b2_ref[...]
    ewt = ewt_ref[...]
    d = jnp.sqrt(jnp.sum(ewt * ewt, axis=0, keepdims=True))
    cut = 0.5 * (jnp.cos(d * (PI / CUTOFF)) + 1.0)
    cut = jnp.where(d <= CUTOFF, cut, 0.0)
    o_ref[...] = _pack_bf16_pairs(w * cut.T, w.shape[1])


def _make_sc_scatter(n_nodes, n_edges, feat, ebase=0):
    nw = NC * NS
    assert n_edges % nw == 0 and ebase % 8 == 0
    epw = n_edges // nw
    k = None
    for cand in range(128, 0, -8):
        if epw % cand or (epw // cand) % 2:
            continue
        per_tile = 2 * cand * feat * (4 + 2 + 4) + epw * 4 + 8192
        if n_nodes * feat * 4 + NS * per_tile <= int(7.8 * 1024 * 1024):
            k = cand
            break
    assert k is not None
    nchunk = epw // k
    nrz = (n_nodes // NS) // 8 * 8
    tail = n_nodes - NS * nrz
    assert tail >= 0 and tail % 8 == 0

    assert nchunk % 2 == 0
    npair = nchunk // 2

    mesh = plsc.VectorSubcoreMesh(core_axis_name="c", subcore_axis_name="s",
                                  num_cores=NC, num_subcores=NS)

    @functools.partial(
        pl.kernel,
        out_type=jax.ShapeDtypeStruct((NC, n_nodes, feat), jnp.float32),
        mesh=mesh,
        compiler_params=pltpu.CompilerParams(needs_layout_passes=False),
        scratch_types=[
            pltpu.VMEM((epw,), jnp.int32),
            pltpu.VMEM((4, k), jnp.int32),
            pltpu.VMEM((2, k, feat), jnp.float32),
            pltpu.VMEM((2, k, feat // 2), jnp.int32),
            pltpu.VMEM((2, k, feat), jnp.float32),
            pltpu.VMEM_SHARED((n_nodes, feat), jnp.float32),
            pltpu.SemaphoreType.DMA,
            pltpu.SemaphoreType.DMA,
            pltpu.SemaphoreType.DMA,
            pltpu.SemaphoreType.DMA,
            pltpu.SemaphoreType.DMA,
            pltpu.SemaphoreType.DMA,
            pltpu.SemaphoreType.DMA,
            pltpu.SemaphoreType.DMA,
        ],
    )
    def sc_kernel(i_hbm, j_hbm, wc_hbm, xw_hbm, out_hbm,
                  jj_v, ii_v, rows_v, w_v, msg_v, agg_sh,
                  g0, g1, w0, w1, i0, i1, s0, s1):
        cid = lax.axis_index("c")
        sid = lax.axis_index("s")
        wid = cid * NS + sid
        gsem = (g0, g1)
        wsem = (w0, w1)
        isem = (i0, i1)
        ssem = (s0, s1)

        pltpu.sync_copy(j_hbm.at[pl.ds(ebase + wid * epw, epw)], jj_v)

        def zero_body(kk, c):
            for cc in range(feat // LANES):
                msg_v[0, kk, pl.ds(cc * LANES, LANES)] = jnp.zeros(
                    (LANES,), jnp.float32)
            return c
        lax.fori_loop(0, k, zero_body, 0)
        zoff = 0
        for sz in [k] * (nrz // k) + ([nrz % k] if nrz % k else []):
            pltpu.sync_copy(msg_v.at[0, pl.ds(0, sz)],
                            agg_sh.at[pl.ds(sid * nrz + zoff, sz)])
            zoff += sz
        if tail:
            @pl.when(sid == 0)
            def _zero_tail():
                pltpu.sync_copy(msg_v.at[0, pl.ds(0, tail)],
                                agg_sh.at[pl.ds(NS * nrz, tail)])
        plsc.subcore_barrier()

        def islot(t, b):
            return 2 * lax.rem(lax.div(t, 2), 2) + b

        def start(t, b):
            off = wid * epw + t * k
            pltpu.async_copy(i_hbm.at[pl.ds(ebase + off, k)],
                             ii_v.at[islot(t, b)], isem[b])
            pltpu.async_copy(xw_hbm.at[jj_v.at[pl.ds(t * k, k)]],
                             rows_v.at[b], gsem[b])
            pltpu.async_copy(wc_hbm.at[pl.ds(off, k)], w_v.at[b], wsem[b])

        def finish(t, b, wait_scatter):
            off = wid * epw + t * k
            pltpu.make_async_copy(xw_hbm.at[jj_v.at[pl.ds(t * k, k)]],
                                  rows_v.at[b], gsem[b]).wait()
            pltpu.make_async_copy(wc_hbm.at[pl.ds(off, k)], w_v.at[b],
                                  wsem[b]).wait()
            if wait_scatter:
                pltpu.make_async_copy(msg_v.at[b],
                                      agg_sh.at[ii_v.at[islot(t, b)]],
                                      ssem[b]).wait()

            def mul_body(kk, c2):
                half = feat // 2
                for cc in range(feat // (2 * LANES)):
                    wbf = plsc.bitcast(w_v[b, kk, pl.ds(cc * LANES, LANES)],
                                       jnp.bfloat16)
                    wa, wb = plsc.unpack(
                        wbf, format=plsc.PackFormat.INTERLEAVED)
                    slo = pl.ds(cc * LANES, LANES)
                    shi = pl.ds(half + cc * LANES, LANES)
                    msg_v[b, kk, slo] = rows_v[b, kk, slo] * wa
                    msg_v[b, kk, shi] = rows_v[b, kk, shi] * wb
                return c2
            lax.fori_loop(0, k, mul_body, 0)
            pltpu.make_async_copy(i_hbm.at[pl.ds(ebase + off, k)],
                                  ii_v.at[islot(t, b)], isem[b]).wait()
            pltpu.async_copy(msg_v.at[b], agg_sh.at[ii_v.at[islot(t, b)]],
                             ssem[b], add=True)

        start(0, 0)
        start(1, 1)
        finish(0, 0, False)
        start(2, 0)
        finish(1, 1, False)
        start(3, 1)

        def pair_body(p, c):
            t = 2 * p
            finish(t, 0, True)
            start(t + 2, 0)
            finish(t + 1, 1, True)
            start(t + 3, 1)
            return c
        lax.fori_loop(1, npair - 1, pair_body, 0)
        finish(nchunk - 2, 0, True)
        finish(nchunk - 1, 1, True)
        pltpu.make_async_copy(msg_v.at[0],
                              agg_sh.at[ii_v.at[islot(nchunk - 2, 0)]],
                              ssem[0]).wait()
        pltpu.make_async_copy(msg_v.at[1],
                              agg_sh.at[ii_v.at[islot(nchunk - 1, 1)]],
                              ssem[1]).wait()

        plsc.subcore_barrier()
        r0 = sid * nrz
        pltpu.sync_copy(agg_sh.at[pl.ds(r0, nrz)],
                        out_hbm.at[cid, pl.ds(r0, nrz)])
        if tail:
            @pl.when(sid == 0)
            def _dump_tail():
                pltpu.sync_copy(agg_sh.at[pl.ds(NS * nrz, tail)],
                                out_hbm.at[cid, pl.ds(NS * nrz, tail)])

    class _SC:
        fn = staticmethod(sc_kernel)
        chunk = k

    return _SC


def kernel(x, edge_index, edge_weight, edge_attr,
           mlp_w1, mlp_b1, mlp_w2, mlp_b2,
           lin1_w, lin2_w, lin2_b):
    n, h = x.shape
    e = edge_index.shape[1]
    g = edge_attr.shape[1]
    f = lin1_w.shape[1]

    bn = 1000
    assert n % bn == 0
    be = 2560
    assert e % be == 0

    xw = pl.pallas_call(
        _xw_body,
        grid=(n // bn,),
        in_specs=[
            pl.BlockSpec((bn, h), lambda i: (i, 0)),
            pl.BlockSpec((h, f), lambda i: (0, 0)),
        ],
        out_specs=pl.BlockSpec((bn, f), lambda i: (i, 0)),
        out_shape=jax.ShapeDtypeStruct((n, f), jnp.float32),
    )(x, lin1_w)

    nblocks = e // be
    splits = []
    lo = 0
    for part in range(NSPLIT):
        hi = nblocks * (part + 1) // NSPLIT
        if hi > lo:
            splits.append((lo, hi - lo))
            lo = hi

    def _edge_filter_call(blo, nblk):
        return pl.pallas_call(
            _edge_filter_body,
            grid=(nblk,),
            in_specs=[
                pl.BlockSpec((g, be), lambda i, blo=blo: (0, i + blo)),
                pl.BlockSpec((3, be), lambda i, blo=blo: (0, i + blo)),
                pl.BlockSpec((g, f), lambda i: (0, 0)),
                pl.BlockSpec((1, f), lambda i: (0, 0)),
                pl.BlockSpec((f, f), lambda i: (0, 0)),
                pl.BlockSpec((1, f), lambda i: (0, 0)),
            ],
            out_specs=pl.BlockSpec((be, f // 2), lambda i: (i, 0)),
            out_shape=jax.ShapeDtypeStruct((nblk * be, f // 2), jnp.int32),
        )(edge_attr.T, edge_weight.T, mlp_w1, mlp_b1.reshape(1, f),
          mlp_w2, mlp_b2.reshape(1, f))

    partials = []
    for blo, nblk in splits:
        wc = _edge_filter_call(blo, nblk)
        sc = _make_sc_scatter(n, nblk * be, f, ebase=blo * be)
        partials.append(sc.fn(edge_index[0], edge_index[1], wc, xw))

    nsp = len(partials)

    def _final_body(x_ref, *rest):
        p_refs = rest[:nsp]
        w_ref, b_ref, o_ref = rest[nsp:]
        agg = p_refs[0][0] + p_refs[0][1]
        for pr in p_refs[1:]:
            agg = agg + pr[0] + pr[1]
        hh = jnp.dot(agg, w_ref[...],
                     preferred_element_type=jnp.float32) + b_ref[...]
        o_ref[...] = x_ref[...] + ssp(hh)

    out = pl.pallas_call(
        _final_body,
        grid=(n // bn,),
        in_specs=[pl.BlockSpec((bn, h), lambda i: (i, 0))]
        + [pl.BlockSpec((2, bn, f), lambda i: (0, i, 0))] * nsp
        + [
            pl.BlockSpec((f, h), lambda i: (0, 0)),
            pl.BlockSpec((1, h), lambda i: (0, 0)),
        ],
        out_specs=pl.BlockSpec((bn, h), lambda i: (i, 0)),
        out_shape=jax.ShapeDtypeStruct((n, h), jnp.float32),
    )(x, *partials, lin2_w, lin2_b.reshape(1, h))

    return out


# speedup vs baseline: 1.0573x; 1.0573x over previous
import functools

import jax
import jax.numpy as jnp
from jax import lax
from jax.experimental import pallas as pl
from jax.experimental.pallas import tpu as pltpu
from jax.experimental.pallas import tpu_sc as plsc

LOG2 = 0.6931471805599453
PI = 3.141592653589793
CUTOFF = 10.0

NC = 2
NS = 16
LANES = 16
NSPLIT = 2


def ssp(v):
    return jnp.maximum(v, 0.0) + jnp.log(1.0 + jnp.exp(-jnp.abs(v))) - LOG2


def _pack_bf16_pairs(y, feat):
    half = feat // 2
    yb = y.astype(jnp.bfloat16)
    lo = lax.bitcast_convert_type(yb[:, :half], jnp.uint16).astype(jnp.uint32)
    hi = lax.bitcast_convert_type(yb[:, half:], jnp.uint16).astype(jnp.uint32)
    return lax.bitcast_convert_type(lo | (hi << 16), jnp.int32)


def _xw_body(x_ref, w_ref, o_ref):
    o_ref[...] = jnp.dot(x_ref[...], w_ref[...],
                         preferred_element_type=jnp.float32)


_COSQ = [1.0, -1 / 2, 1 / 24, -1 / 720, 1 / 40320, -1 / 3628800,
         1 / 479001600, -1 / 87178291200]


def _edge_filter_body(eat_ref, ewt_ref, w1_ref, b1_ref, w2_ref, b2_ref, o_ref):
    h1 = lax.dot_general(eat_ref[...].astype(jnp.bfloat16),
                         w1_ref[...].astype(jnp.bfloat16),
                         (((0,), (0,)), ((), ())),
                         preferred_element_type=jnp.float32) + b1_ref[...]
    h1 = ssp(h1)
    w = jnp.dot(h1.astype(jnp.bfloat16), w2_ref[...].astype(jnp.bfloat16),
                preferred_element_type=jnp.float32) + b2_ref[...]
    ewt = ewt_ref[...]
    d2 = jnp.sum(ewt * ewt, axis=0, keepdims=True)
    u = d2 * ((PI / CUTOFF) * (PI / CUTOFF))
    c = _COSQ[-1]
    for coef in reversed(_COSQ[:-1]):
        c = c * u + coef
    cut = 0.5 * (c + 1.0)
    cut = jnp.where(d2 <= CUTOFF * CUTOFF, cut, 0.0)
    o_ref[...] = _pack_bf16_pairs(w * cut.T, w.shape[1])


def _make_sc_scatter(n_nodes, n_edges, feat, ebase=0):
    nw = NC * NS
    assert n_edges % nw == 0 and ebase % 8 == 0
    epw = n_edges // nw
    k = None
    for cand in range(128, 0, -8):
        if epw % cand or (epw // cand) % 2:
            continue
        per_tile = 2 * cand * feat * (4 + 2 + 4) + epw * 4 + 8192
        if n_nodes * feat * 4 + NS * per_tile <= int(7.8 * 1024 * 1024):
            k = cand
            break
    assert k is not None
    nchunk = epw // k
    nrz = (n_nodes // NS) // 8 * 8
    tail = n_nodes - NS * nrz
    assert tail >= 0 and tail % 8 == 0

    assert nchunk % 2 == 0
    npair = nchunk // 2

    mesh = plsc.VectorSubcoreMesh(core_axis_name="c", subcore_axis_name="s",
                                  num_cores=NC, num_subcores=NS)

    @functools.partial(
        pl.kernel,
        out_type=jax.ShapeDtypeStruct((NC, n_nodes, feat), jnp.float32),
        mesh=mesh,
        compiler_params=pltpu.CompilerParams(needs_layout_passes=False),
        scratch_types=[
            pltpu.VMEM((epw,), jnp.int32),
            pltpu.VMEM((4, k), jnp.int32),
            pltpu.VMEM((2, k, feat), jnp.float32),
            pltpu.VMEM((2, k, feat // 2), jnp.int32),
            pltpu.VMEM((2, k, feat), jnp.float32),
            pltpu.VMEM_SHARED((n_nodes, feat), jnp.float32),
            pltpu.SemaphoreType.DMA,
            pltpu.SemaphoreType.DMA,
            pltpu.SemaphoreType.DMA,
            pltpu.SemaphoreType.DMA,
            pltpu.SemaphoreType.DMA,
            pltpu.SemaphoreType.DMA,
            pltpu.SemaphoreType.DMA,
            pltpu.SemaphoreType.DMA,
        ],
    )
    def sc_kernel(i_hbm, j_hbm, wc_hbm, xw_hbm, out_hbm,
                  jj_v, ii_v, rows_v, w_v, msg_v, agg_sh,
                  g0, g1, w0, w1, i0, i1, s0, s1):
        cid = lax.axis_index("c")
        sid = lax.axis_index("s")
        wid = cid * NS + sid
        gsem = (g0, g1)
        wsem = (w0, w1)
        isem = (i0, i1)
        ssem = (s0, s1)

        pltpu.sync_copy(j_hbm.at[pl.ds(ebase + wid * epw, epw)], jj_v)

        def zero_body(kk, c):
            for cc in range(feat // LANES):
                msg_v[0, kk, pl.ds(cc * LANES, LANES)] = jnp.zeros(
                    (LANES,), jnp.float32)
            return c
        lax.fori_loop(0, k, zero_body, 0)
        zoff = 0
        for sz in [k] * (nrz // k) + ([nrz % k] if nrz % k else []):
            pltpu.sync_copy(msg_v.at[0, pl.ds(0, sz)],
                            agg_sh.at[pl.ds(sid * nrz + zoff, sz)])
            zoff += sz
        if tail:
            @pl.when(sid == 0)
            def _zero_tail():
                pltpu.sync_copy(msg_v.at[0, pl.ds(0, tail)],
                                agg_sh.at[pl.ds(NS * nrz, tail)])
        plsc.subcore_barrier()

        def islot(t, b):
            return 2 * lax.rem(lax.div(t, 2), 2) + b

        def start(t, b):
            off = wid * epw + t * k
            pltpu.async_copy(i_hbm.at[pl.ds(ebase + off, k)],
                             ii_v.at[islot(t, b)], isem[b])
            pltpu.async_copy(xw_hbm.at[jj_v.at[pl.ds(t * k, k)]],
                             rows_v.at[b], gsem[b])
            pltpu.async_copy(wc_hbm.at[pl.ds(off, k)], w_v.at[b], wsem[b])

        def finish(t, b, wait_scatter):
            off = wid * epw + t * k
            pltpu.make_async_copy(xw_hbm.at[jj_v.at[pl.ds(t * k, k)]],
                                  rows_v.at[b], gsem[b]).wait()
            pltpu.make_async_copy(wc_hbm.at[pl.ds(off, k)], w_v.at[b],
                                  wsem[b]).wait()
            if wait_scatter:
                pltpu.make_async_copy(msg_v.at[b],
                                      agg_sh.at[ii_v.at[islot(t, b)]],
                                      ssem[b]).wait()

            def mul_body(kk, c2):
                half = feat // 2
                for cc in range(feat // (2 * LANES)):
                    wbf = plsc.bitcast(w_v[b, kk, pl.ds(cc * LANES, LANES)],
                                       jnp.bfloat16)
                    wa, wb = plsc.unpack(
                        wbf, format=plsc.PackFormat.INTERLEAVED)
                    slo = pl.ds(cc * LANES, LANES)
                    shi = pl.ds(half + cc * LANES, LANES)
                    msg_v[b, kk, slo] = rows_v[b, kk, slo] * wa
                    msg_v[b, kk, shi] = rows_v[b, kk, shi] * wb
                return c2
            lax.fori_loop(0, k, mul_body, 0)
            pltpu.make_async_copy(i_hbm.at[pl.ds(ebase + off, k)],
                                  ii_v.at[islot(t, b)], isem[b]).wait()
            pltpu.async_copy(msg_v.at[b], agg_sh.at[ii_v.at[islot(t, b)]],
                             ssem[b], add=True)

        start(0, 0)
        start(1, 1)
        finish(0, 0, False)
        start(2, 0)
        finish(1, 1, False)
        start(3, 1)

        def pair_body(p, c):
            t = 2 * p
            finish(t, 0, True)
            start(t + 2, 0)
            finish(t + 1, 1, True)
            start(t + 3, 1)
            return c
        lax.fori_loop(1, npair - 1, pair_body, 0)
        finish(nchunk - 2, 0, True)
        finish(nchunk - 1, 1, True)
        pltpu.make_async_copy(msg_v.at[0],
                              agg_sh.at[ii_v.at[islot(nchunk - 2, 0)]],
                              ssem[0]).wait()
        pltpu.make_async_copy(msg_v.at[1],
                              agg_sh.at[ii_v.at[islot(nchunk - 1, 1)]],
                              ssem[1]).wait()

        plsc.subcore_barrier()
        r0 = sid * nrz
        pltpu.sync_copy(agg_sh.at[pl.ds(r0, nrz)],
                        out_hbm.at[cid, pl.ds(r0, nrz)])
        if tail:
            @pl.when(sid == 0)
            def _dump_tail():
                pltpu.sync_copy(agg_sh.at[pl.ds(NS * nrz, tail)],
                                out_hbm.at[cid, pl.ds(NS * nrz, tail)])

    class _SC:
        fn = staticmethod(sc_kernel)
        chunk = k

    return _SC


def kernel(x, edge_index, edge_weight, edge_attr,
           mlp_w1, mlp_b1, mlp_w2, mlp_b2,
           lin1_w, lin2_w, lin2_b):
    n, h = x.shape
    e = edge_index.shape[1]
    g = edge_attr.shape[1]
    f = lin1_w.shape[1]

    bn = 1000
    assert n % bn == 0
    be = 2560
    assert e % be == 0

    xw = pl.pallas_call(
        _xw_body,
        grid=(n // bn,),
        in_specs=[
            pl.BlockSpec((bn, h), lambda i: (i, 0)),
            pl.BlockSpec((h, f), lambda i: (0, 0)),
        ],
        out_specs=pl.BlockSpec((bn, f), lambda i: (i, 0)),
        out_shape=jax.ShapeDtypeStruct((n, f), jnp.float32),
    )(x, lin1_w)

    nblocks = e // be
    splits = []
    lo = 0
    for part in range(NSPLIT):
        hi = nblocks * (part + 1) // NSPLIT
        if hi > lo:
            splits.append((lo, hi - lo))
            lo = hi

    def _edge_filter_call(blo, nblk):
        return pl.pallas_call(
            _edge_filter_body,
            grid=(nblk,),
            in_specs=[
                pl.BlockSpec((g, be), lambda i, blo=blo: (0, i + blo)),
                pl.BlockSpec((3, be), lambda i, blo=blo: (0, i + blo)),
                pl.BlockSpec((g, f), lambda i: (0, 0)),
                pl.BlockSpec((1, f), lambda i: (0, 0)),
                pl.BlockSpec((f, f), lambda i: (0, 0)),
                pl.BlockSpec((1, f), lambda i: (0, 0)),
            ],
            out_specs=pl.BlockSpec((be, f // 2), lambda i: (i, 0)),
            out_shape=jax.ShapeDtypeStruct((nblk * be, f // 2), jnp.int32),
        )(edge_attr.T, edge_weight.T, mlp_w1, mlp_b1.reshape(1, f),
          mlp_w2, mlp_b2.reshape(1, f))

    partials = []
    for blo, nblk in splits:
        wc = _edge_filter_call(blo, nblk)
        sc = _make_sc_scatter(n, nblk * be, f, ebase=blo * be)
        partials.append(sc.fn(edge_index[0], edge_index[1], wc, xw))

    nsp = len(partials)

    def _final_body(x_ref, *rest):
        p_refs = rest[:nsp]
        w_ref, b_ref, o_ref = rest[nsp:]
        agg = p_refs[0][0] + p_refs[0][1]
        for pr in p_refs[1:]:
            agg = agg + pr[0] + pr[1]
        hh = jnp.dot(agg, w_ref[...],
                     preferred_element_type=jnp.float32) + b_ref[...]
        o_ref[...] = x_ref[...] + ssp(hh)

    out = pl.pallas_call(
        _final_body,
        grid=(n // bn,),
        in_specs=[pl.BlockSpec((bn, h), lambda i: (i, 0))]
        + [pl.BlockSpec((2, bn, f), lambda i: (0, i, 0))] * nsp
        + [
            pl.BlockSpec((f, h), lambda i: (0, 0)),
            pl.BlockSpec((1, h), lambda i: (0, 0)),
        ],
        out_specs=pl.BlockSpec((bn, h), lambda i: (i, 0)),
        out_shape=jax.ShapeDtypeStruct((n, h), jnp.float32),
    )(x, *partials, lin2_w, lin2_b.reshape(1, h))

    return out


# speedup vs baseline: 1.0843x; 1.0256x over previous
import functools

import jax
import jax.numpy as jnp
from jax import lax
from jax.experimental import pallas as pl
from jax.experimental.pallas import tpu as pltpu
from jax.experimental.pallas import tpu_sc as plsc

LOG2 = 0.6931471805599453
PI = 3.141592653589793
CUTOFF = 10.0

NC = 2
NS = 16
LANES = 16
NSPLIT = 3


def ssp(v):
    return jnp.maximum(v, 0.0) + jnp.log(1.0 + jnp.exp(-jnp.abs(v))) - LOG2


def _pack_bf16_pairs(y, feat):
    half = feat // 2
    yb = y.astype(jnp.bfloat16)
    lo = lax.bitcast_convert_type(yb[:, :half], jnp.uint16).astype(jnp.uint32)
    hi = lax.bitcast_convert_type(yb[:, half:], jnp.uint16).astype(jnp.uint32)
    return lax.bitcast_convert_type(lo | (hi << 16), jnp.int32)


def _xw_body(x_ref, w_ref, o_ref):
    o_ref[...] = jnp.dot(x_ref[...], w_ref[...],
                         preferred_element_type=jnp.float32)


_COSQ = [1.0, -1 / 2, 1 / 24, -1 / 720, 1 / 40320, -1 / 3628800,
         1 / 479001600, -1 / 87178291200]


def _edge_filter_body(eat_ref, ewt_ref, w1_ref, b1_ref, w2_ref, b2_ref, o_ref):
    h1 = lax.dot_general(eat_ref[...].astype(jnp.bfloat16),
                         w1_ref[...].astype(jnp.bfloat16),
                         (((0,), (0,)), ((), ())),
                         preferred_element_type=jnp.float32) + b1_ref[...]
    h1 = ssp(h1)
    w = jnp.dot(h1.astype(jnp.bfloat16), w2_ref[...].astype(jnp.bfloat16),
                preferred_element_type=jnp.float32) + b2_ref[...]
    ewt = ewt_ref[...]
    d2 = jnp.sum(ewt * ewt, axis=0, keepdims=True)
    u = d2 * ((PI / CUTOFF) * (PI / CUTOFF))
    c = _COSQ[-1]
    for coef in reversed(_COSQ[:-1]):
        c = c * u + coef
    cut = 0.5 * (c + 1.0)
    cut = jnp.where(d2 <= CUTOFF * CUTOFF, cut, 0.0)
    o_ref[...] = _pack_bf16_pairs(w * cut.T, w.shape[1])


def _make_sc_scatter(n_nodes, n_edges, feat, ebase=0):
    nw = NC * NS
    assert n_edges % nw == 0 and ebase % 8 == 0
    epw = n_edges // nw
    k = None
    for cand in range(128, 0, -8):
        if epw % cand or (epw // cand) % 2:
            continue
        per_tile = 2 * cand * feat * (4 + 2 + 4) + epw * 4 + 8192
        if n_nodes * feat * 4 + NS * per_tile <= int(7.8 * 1024 * 1024):
            k = cand
            break
    assert k is not None
    nchunk = epw // k
    nrz = (n_nodes // NS) // 8 * 8
    tail = n_nodes - NS * nrz
    assert tail >= 0 and tail % 8 == 0

    assert nchunk % 2 == 0
    npair = nchunk // 2

    mesh = plsc.VectorSubcoreMesh(core_axis_name="c", subcore_axis_name="s",
                                  num_cores=NC, num_subcores=NS)

    @functools.partial(
        pl.kernel,
        out_type=jax.ShapeDtypeStruct((NC, n_nodes, feat), jnp.float32),
        mesh=mesh,
        compiler_params=pltpu.CompilerParams(needs_layout_passes=False),
        scratch_types=[
            pltpu.VMEM((epw,), jnp.int32),
            pltpu.VMEM((4, k), jnp.int32),
            pltpu.VMEM((2, k, feat), jnp.float32),
            pltpu.VMEM((2, k, feat // 2), jnp.int32),
            pltpu.VMEM((2, k, feat), jnp.float32),
            pltpu.VMEM_SHARED((n_nodes, feat), jnp.float32),
            pltpu.SemaphoreType.DMA,
            pltpu.SemaphoreType.DMA,
            pltpu.SemaphoreType.DMA,
            pltpu.SemaphoreType.DMA,
            pltpu.SemaphoreType.DMA,
            pltpu.SemaphoreType.DMA,
            pltpu.SemaphoreType.DMA,
            pltpu.SemaphoreType.DMA,
        ],
    )
    def sc_kernel(i_hbm, j_hbm, wc_hbm, xw_hbm, out_hbm,
                  jj_v, ii_v, rows_v, w_v, msg_v, agg_sh,
                  g0, g1, w0, w1, i0, i1, s0, s1):
        cid = lax.axis_index("c")
        sid = lax.axis_index("s")
        wid = cid * NS + sid
        gsem = (g0, g1)
        wsem = (w0, w1)
        isem = (i0, i1)
        ssem = (s0, s1)

        pltpu.sync_copy(j_hbm.at[pl.ds(ebase + wid * epw, epw)], jj_v)

        def zero_body(kk, c):
            for cc in range(feat // LANES):
                msg_v[0, kk, pl.ds(cc * LANES, LANES)] = jnp.zeros(
                    (LANES,), jnp.float32)
            return c
        lax.fori_loop(0, k, zero_body, 0)
        zoff = 0
        for sz in [k] * (nrz // k) + ([nrz % k] if nrz % k else []):
            pltpu.sync_copy(msg_v.at[0, pl.ds(0, sz)],
                            agg_sh.at[pl.ds(sid * nrz + zoff, sz)])
            zoff += sz
        if tail:
            @pl.when(sid == 0)
            def _zero_tail():
                pltpu.sync_copy(msg_v.at[0, pl.ds(0, tail)],
                                agg_sh.at[pl.ds(NS * nrz, tail)])
        plsc.subcore_barrier()

        def islot(t, b):
            return 2 * lax.rem(lax.div(t, 2), 2) + b

        def start(t, b):
            off = wid * epw + t * k
            pltpu.async_copy(i_hbm.at[pl.ds(ebase + off, k)],
                             ii_v.at[islot(t, b)], isem[b])
            pltpu.async_copy(xw_hbm.at[jj_v.at[pl.ds(t * k, k)]],
                             rows_v.at[b], gsem[b])
            pltpu.async_copy(wc_hbm.at[pl.ds(off, k)], w_v.at[b], wsem[b])

        def finish(t, b, wait_scatter):
            off = wid * epw + t * k
            pltpu.make_async_copy(xw_hbm.at[jj_v.at[pl.ds(t * k, k)]],
                                  rows_v.at[b], gsem[b]).wait()
            pltpu.make_async_copy(wc_hbm.at[pl.ds(off, k)], w_v.at[b],
                                  wsem[b]).wait()
            if wait_scatter:
                pltpu.make_async_copy(msg_v.at[b],
                                      agg_sh.at[ii_v.at[islot(t, b)]],
                                      ssem[b]).wait()

            def mul_body(kk, c2):
                half = feat // 2
                for cc in range(feat // (2 * LANES)):
                    wbf = plsc.bitcast(w_v[b, kk, pl.ds(cc * LANES, LANES)],
                                       jnp.bfloat16)
                    wa, wb = plsc.unpack(
                        wbf, format=plsc.PackFormat.INTERLEAVED)
                    slo = pl.ds(cc * LANES, LANES)
                    shi = pl.ds(half + cc * LANES, LANES)
                    msg_v[b, kk, slo] = rows_v[b, kk, slo] * wa
                    msg_v[b, kk, shi] = rows_v[b, kk, shi] * wb
                return c2
            lax.fori_loop(0, k, mul_body, 0)
            pltpu.make_async_copy(i_hbm.at[pl.ds(ebase + off, k)],
                                  ii_v.at[islot(t, b)], isem[b]).wait()
            pltpu.async_copy(msg_v.at[b], agg_sh.at[ii_v.at[islot(t, b)]],
                             ssem[b], add=True)

        start(0, 0)
        start(1, 1)
        finish(0, 0, False)
        start(2, 0)
        finish(1, 1, False)
        start(3, 1)

        def pair_body(p, c):
            t = 2 * p
            finish(t, 0, True)
            start(t + 2, 0)
            finish(t + 1, 1, True)
            start(t + 3, 1)
            return c
        lax.fori_loop(1, npair - 1, pair_body, 0)
        finish(nchunk - 2, 0, True)
        finish(nchunk - 1, 1, True)
        pltpu.make_async_copy(msg_v.at[0],
                              agg_sh.at[ii_v.at[islot(nchunk - 2, 0)]],
                              ssem[0]).wait()
        pltpu.make_async_copy(msg_v.at[1],
                              agg_sh.at[ii_v.at[islot(nchunk - 1, 1)]],
                              ssem[1]).wait()

        plsc.subcore_barrier()
        r0 = sid * nrz
        pltpu.sync_copy(agg_sh.at[pl.ds(r0, nrz)],
                        out_hbm.at[cid, pl.ds(r0, nrz)])
        if tail:
            @pl.when(sid == 0)
            def _dump_tail():
                pltpu.sync_copy(agg_sh.at[pl.ds(NS * nrz, tail)],
                                out_hbm.at[cid, pl.ds(NS * nrz, tail)])

    class _SC:
        fn = staticmethod(sc_kernel)
        chunk = k

    return _SC


def kernel(x, edge_index, edge_weight, edge_attr,
           mlp_w1, mlp_b1, mlp_w2, mlp_b2,
           lin1_w, lin2_w, lin2_b):
    n, h = x.shape
    e = edge_index.shape[1]
    g = edge_attr.shape[1]
    f = lin1_w.shape[1]

    bn = 1000
    assert n % bn == 0
    be = 2560
    assert e % be == 0

    xw = pl.pallas_call(
        _xw_body,
        grid=(n // bn,),
        in_specs=[
            pl.BlockSpec((bn, h), lambda i: (i, 0)),
            pl.BlockSpec((h, f), lambda i: (0, 0)),
        ],
        out_specs=pl.BlockSpec((bn, f), lambda i: (i, 0)),
        out_shape=jax.ShapeDtypeStruct((n, f), jnp.float32),
    )(x, lin1_w)

    nblocks = e // be
    splits = []
    lo = 0
    for part in range(NSPLIT):
        hi = nblocks * (part + 1) // NSPLIT
        if hi > lo:
            splits.append((lo, hi - lo))
            lo = hi

    def _edge_filter_call(blo, nblk):
        return pl.pallas_call(
            _edge_filter_body,
            grid=(nblk,),
            in_specs=[
                pl.BlockSpec((g, be), lambda i, blo=blo: (0, i + blo)),
                pl.BlockSpec((3, be), lambda i, blo=blo: (0, i + blo)),
                pl.BlockSpec((g, f), lambda i: (0, 0)),
                pl.BlockSpec((1, f), lambda i: (0, 0)),
                pl.BlockSpec((f, f), lambda i: (0, 0)),
                pl.BlockSpec((1, f), lambda i: (0, 0)),
            ],
            out_specs=pl.BlockSpec((be, f // 2), lambda i: (i, 0)),
            out_shape=jax.ShapeDtypeStruct((nblk * be, f // 2), jnp.int32),
        )(edge_attr.T, edge_weight.T, mlp_w1, mlp_b1.reshape(1, f),
          mlp_w2, mlp_b2.reshape(1, f))

    partials = []
    for blo, nblk in splits:
        wc = _edge_filter_call(blo, nblk)
        sc = _make_sc_scatter(n, nblk * be, f, ebase=blo * be)
        partials.append(sc.fn(edge_index[0], edge_index[1], wc, xw))

    nsp = len(partials)

    def _final_body(x_ref, *rest):
        p_refs = rest[:nsp]
        w_ref, b_ref, o_ref = rest[nsp:]
        agg = p_refs[0][0] + p_refs[0][1]
        for pr in p_refs[1:]:
            agg = agg + pr[0] + pr[1]
        hh = jnp.dot(agg, w_ref[...],
                     preferred_element_type=jnp.float32) + b_ref[...]
        o_ref[...] = x_ref[...] + ssp(hh)

    out = pl.pallas_call(
        _final_body,
        grid=(n // bn,),
        in_specs=[pl.BlockSpec((bn, h), lambda i: (i, 0))]
        + [pl.BlockSpec((2, bn, f), lambda i: (0, i, 0))] * nsp
        + [
            pl.BlockSpec((f, h), lambda i: (0, 0)),
            pl.BlockSpec((1, h), lambda i: (0, 0)),
        ],
        out_specs=pl.BlockSpec((bn, h), lambda i: (i, 0)),
        out_shape=jax.ShapeDtypeStruct((n, h), jnp.float32),
    )(x, *partials, lin2_w, lin2_b.reshape(1, h))

    return out
